# baseline (speedup 1.0000x reference)
import functools

import jax
import jax.numpy as jnp
from jax import lax
from jax.experimental import pallas as pl
from jax.experimental.pallas import tpu as pltpu
from jax.experimental.pallas import tpu_sc as plsc

_B = 16384
_L = 20
_NC = 2
_NS = 16
_NW = _NC * _NS
_LANES = 16
_ROWS_PER_W = _B // _NW
_CHUNK_ROWS = 128
_NCHUNK = _ROWS_PER_W // _CHUNK_ROWS
_TOL = 0.5
_TSTRIDE = _ROWS_PER_W + 1

_mesh = plsc.VectorSubcoreMesh(
    core_axis_name="c", subcore_axis_name="s",
    num_cores=_NC, num_subcores=_NS)


@functools.partial(
    pl.kernel,
    out_type=jax.ShapeDtypeStruct((_NW, 2, _LANES), jnp.float32),
    mesh=_mesh,
    compiler_params=pltpu.CompilerParams(needs_layout_passes=False),
    scratch_types=[
        pltpu.VMEM((_CHUNK_ROWS, _L), jnp.int32),
        pltpu.VMEM((_CHUNK_ROWS, _L), jnp.float32),
        pltpu.VMEM((120,), jnp.float32),
        pltpu.VMEM((_L * _TSTRIDE,), jnp.float32),
        pltpu.VMEM((2, _LANES), jnp.float32),
    ],
)
def _sc_charge_loss(pw_hbm, frac_hbm, table_hbm, out_hbm,
                    pw_v, frac_v, table_v, charge_t, out_v):
    wid = lax.axis_index("s") * _NC + lax.axis_index("c")
    rowbase = wid * _ROWS_PER_W
    pltpu.sync_copy(table_hbm, table_v)

    iota = lax.iota(jnp.int32, _LANES)

    for ck in range(_NCHUNK):
        pltpu.sync_copy(
            pw_hbm.at[pl.ds(rowbase + ck * _CHUNK_ROWS, _CHUNK_ROWS), :],
            pw_v)
        pltpu.sync_copy(
            frac_hbm.at[pl.ds(rowbase + ck * _CHUNK_ROWS, _CHUNK_ROWS), :],
            frac_v)

        def phase1(i, carry):
            e = i * _LANES + iota
            r = e // _L
            c = e % _L
            w = plsc.load_gather(pw_v, [r, c])
            f = plsc.load_gather(frac_v, [r, c])
            idx = jnp.minimum(w & 127, 119)
            mf = (w >> 7).astype(jnp.float32)
            ox = plsc.load_gather(table_v, [idx])
            tidx = c * _TSTRIDE + (ck * _CHUNK_ROWS) + r
            plsc.store_scatter(charge_t, [tidx], f * mf * ox)
            return carry
        lax.fori_loop(0, _CHUNK_ROWS * _L // _LANES, phase1, 0)

    def phase2(j, carry):
        loss_acc, abs_acc = carry
        r = j * _LANES
        tc = charge_t[pl.ds(r, _LANES)]
        for l in range(1, _L):
            tc = tc + charge_t[pl.ds(l * _TSTRIDE + r, _LANES)]
        a = jnp.abs(tc)
        ex = jnp.maximum(a - _TOL, 0.0)
        e2 = jnp.exp(2.0 * ex)
        t = 1.0 - 2.0 / (e2 + 1.0)
        return loss_acc + t, abs_acc + a

    zero = jnp.zeros((_LANES,), jnp.float32)
    loss_acc, abs_acc = lax.fori_loop(
        0, _ROWS_PER_W // _LANES, phase2, (zero, zero))

    out_v[0, :] = loss_acc * (1.0 / _B)
    out_v[1, :] = abs_acc * (1.0 / _B)
    pltpu.sync_copy(out_v, out_hbm.at[wid])


def kernel(element_indices, element_fractions, element_mask, oxidation_states):
    pw = (element_indices.astype(jnp.int32)
          | (element_mask.astype(jnp.int32) << 7))
    partials = _sc_charge_loss(pw, element_fractions, oxidation_states)
    charge_balance_loss = jnp.sum(partials[:, 0, :])
    mean_charge_imbalance = jnp.sum(partials[:, 1, :])
    return (charge_balance_loss, mean_charge_imbalance)


# speedup vs baseline: 1.0462x; 1.0462x over previous
import functools

import jax
import jax.numpy as jnp
from jax import lax
from jax.experimental import pallas as pl
from jax.experimental.pallas import tpu as pltpu
from jax.experimental.pallas import tpu_sc as plsc

_B = 16384
_L = 20
_NC = 2
_NS = 16
_NW = _NC * _NS
_LANES = 16
_ROWS_PER_W = _B // _NW
_ELEMS_PER_W = _ROWS_PER_W * _L
_CC = 128
_CROWS = _B * _L // _CC
_CROWS_PER_W = _CROWS // _NW
_TOL = 0.5
_TSTRIDE = _ROWS_PER_W + 1
_UNROLL = 4

_mesh = plsc.VectorSubcoreMesh(
    core_axis_name="c", subcore_axis_name="s",
    num_cores=_NC, num_subcores=_NS)


@functools.partial(
    pl.kernel,
    out_type=jax.ShapeDtypeStruct((_NW, 2, _LANES), jnp.float32),
    mesh=_mesh,
    compiler_params=pltpu.CompilerParams(needs_layout_passes=False),
    scratch_types=[
        pltpu.VMEM((_CROWS_PER_W, _CC), jnp.int32),
        pltpu.VMEM((_CROWS_PER_W, _CC), jnp.float32),
        pltpu.VMEM((120,), jnp.float32),
        pltpu.VMEM((_L * _TSTRIDE,), jnp.float32),
        pltpu.VMEM((2, _LANES), jnp.float32),
    ],
)
def _sc_charge_loss(pw_hbm, frac_hbm, table_hbm, out_hbm,
                    pw_v, frac_v, table_v, charge_t, out_v):
    wid = lax.axis_index("s") * _NC + lax.axis_index("c")
    pltpu.sync_copy(table_hbm, table_v)
    pltpu.sync_copy(pw_hbm.at[pl.ds(wid * _CROWS_PER_W, _CROWS_PER_W), :],
                    pw_v)
    pltpu.sync_copy(frac_hbm.at[pl.ds(wid * _CROWS_PER_W, _CROWS_PER_W), :],
                    frac_v)

    iota = lax.iota(jnp.int32, _LANES)

    def phase1(i, carry):
        for u in range(_UNROLL):
            ic = i * _UNROLL + u
            r = ic // (_CC // _LANES)
            c0 = (ic % (_CC // _LANES)) * _LANES
            w = pw_v[r, pl.ds(c0, _LANES)]
            f = frac_v[r, pl.ds(c0, _LANES)]
            idx = jnp.minimum(w & 127, 119)
            mf = (w >> 7).astype(jnp.float32)
            ox = plsc.load_gather(table_v, [idx])
            e = ic * _LANES + iota
            tidx = (e % _L) * _TSTRIDE + e // _L
            plsc.store_scatter(charge_t, [tidx], f * mf * ox)
        return carry
    lax.fori_loop(0, _ELEMS_PER_W // (_LANES * _UNROLL), phase1, 0)

    def phase2(j, carry):
        loss_acc, abs_acc = carry
        r = j * _LANES
        tc = charge_t[pl.ds(r, _LANES)]
        for l in range(1, _L):
            tc = tc + charge_t[pl.ds(l * _TSTRIDE + r, _LANES)]
        a = jnp.abs(tc)
        ex = jnp.maximum(a - _TOL, 0.0)
        e2 = jnp.exp(2.0 * ex)
        t = 1.0 - 2.0 / (e2 + 1.0)
        return loss_acc + t, abs_acc + a

    zero = jnp.zeros((_LANES,), jnp.float32)
    loss_acc, abs_acc = lax.fori_loop(
        0, _ROWS_PER_W // _LANES, phase2, (zero, zero))

    out_v[0, :] = loss_acc * (1.0 / _B)
    out_v[1, :] = abs_acc * (1.0 / _B)
    pltpu.sync_copy(out_v, out_hbm.at[wid])


def kernel(element_indices, element_fractions, element_mask, oxidation_states):
    pw = (element_indices.astype(jnp.int32)
          | (element_mask.astype(jnp.int32) << 7)).reshape(_CROWS, _CC)
    ef = element_fractions.reshape(_CROWS, _CC)
    partials = _sc_charge_loss(pw, ef, oxidation_states)
    charge_balance_loss = jnp.sum(partials[:, 0, :])
    mean_charge_imbalance = jnp.sum(partials[:, 1, :])
    return (charge_balance_loss, mean_charge_imbalance)


# speedup vs baseline: 2.3823x; 2.2771x over previous
import functools

import jax
import jax.numpy as jnp
from jax import lax
from jax.experimental import pallas as pl
from jax.experimental.pallas import tpu as pltpu
from jax.experimental.pallas import tpu_sc as plsc

_B = 16384
_L = 20
_NC = 2
_NS = 16
_NW = _NC * _NS
_LANES = 16
_SAMPLES_PER_W = _B // _NW
_TOL = 0.5

_mesh = plsc.VectorSubcoreMesh(
    core_axis_name="c", subcore_axis_name="s",
    num_cores=_NC, num_subcores=_NS)


@functools.partial(
    pl.kernel,
    out_type=jax.ShapeDtypeStruct((_NW, 2, _LANES), jnp.float32),
    mesh=_mesh,
    compiler_params=pltpu.CompilerParams(needs_layout_passes=False),
    scratch_types=[
        pltpu.VMEM((_L, _SAMPLES_PER_W), jnp.int32),
        pltpu.VMEM((_L, _SAMPLES_PER_W), jnp.float32),
        pltpu.VMEM((120,), jnp.float32),
        pltpu.VMEM((2, _LANES), jnp.float32),
    ],
)
def _sc_charge_loss(pw_hbm, frac_hbm, table_hbm, out_hbm,
                    pw_v, frac_v, table_v, out_v):
    wid = lax.axis_index("s") * _NC + lax.axis_index("c")
    base = wid * _SAMPLES_PER_W
    pltpu.sync_copy(table_hbm, table_v)
    pltpu.sync_copy(pw_hbm.at[:, pl.ds(base, _SAMPLES_PER_W)], pw_v)
    pltpu.sync_copy(frac_hbm.at[:, pl.ds(base, _SAMPLES_PER_W)], frac_v)

    def body(g, carry):
        loss_acc, abs_acc = carry
        c0 = g * _LANES
        tc = jnp.zeros((_LANES,), jnp.float32)
        for l in range(_L):
            w = pw_v[l, pl.ds(c0, _LANES)]
            f = frac_v[l, pl.ds(c0, _LANES)]
            idx = jnp.minimum(w & 127, 119)
            mf = (w >> 7).astype(jnp.float32)
            ox = plsc.load_gather(table_v, [idx])
            tc = tc + f * mf * ox
        a = jnp.abs(tc)
        ex = jnp.maximum(a - _TOL, 0.0)
        e2 = jnp.exp(2.0 * ex)
        t = 1.0 - 2.0 / (e2 + 1.0)
        return loss_acc + t, abs_acc + a

    zero = jnp.zeros((_LANES,), jnp.float32)
    loss_acc, abs_acc = lax.fori_loop(
        0, _SAMPLES_PER_W // _LANES, body, (zero, zero))

    out_v[0, :] = loss_acc * (1.0 / _B)
    out_v[1, :] = abs_acc * (1.0 / _B)
    pltpu.sync_copy(out_v, out_hbm.at[wid])


def kernel(element_indices, element_fractions, element_mask, oxidation_states):
    pw = (element_indices.astype(jnp.int32)
          | (element_mask.astype(jnp.int32) << 7)).T
    ef = element_fractions.T
    partials = _sc_charge_loss(pw, ef, oxidation_states)
    charge_balance_loss = jnp.sum(partials[:, 0, :])
    mean_charge_imbalance = jnp.sum(partials[:, 1, :])
    return (charge_balance_loss, mean_charge_imbalance)


# speedup vs baseline: 2.4776x; 1.0400x over previous
import functools

import jax
import jax.numpy as jnp
from jax import lax
from jax.experimental import pallas as pl
from jax.experimental.pallas import tpu as pltpu
from jax.experimental.pallas import tpu_sc as plsc

_B = 16384
_L = 20
_NC = 2
_NS = 16
_NW = _NC * _NS
_LANES = 16
_SAMPLES_PER_W = _B // _NW
_TOL = 0.5

_mesh = plsc.VectorSubcoreMesh(
    core_axis_name="c", subcore_axis_name="s",
    num_cores=_NC, num_subcores=_NS)


@functools.partial(
    pl.kernel,
    out_type=jax.ShapeDtypeStruct((_NW, 2, _LANES), jnp.float32),
    mesh=_mesh,
    compiler_params=pltpu.CompilerParams(needs_layout_passes=False),
    scratch_types=[
        pltpu.VMEM((_L, _SAMPLES_PER_W), jnp.int32),
        pltpu.VMEM((_L, _SAMPLES_PER_W), jnp.float32),
        pltpu.VMEM((120,), jnp.float32),
        pltpu.VMEM((256,), jnp.float32),
        pltpu.VMEM((2, _LANES), jnp.float32),
        pltpu.SemaphoreType.DMA,
        pltpu.SemaphoreType.DMA,
    ],
)
def _sc_charge_loss(pw_hbm, frac_hbm, table_hbm, out_hbm,
                    pw_v, frac_v, table_v, t256_v, out_v, sem1, sem2):
    wid = lax.axis_index("s") * _NC + lax.axis_index("c")
    base = wid * _SAMPLES_PER_W
    cp1 = pltpu.async_copy(pw_hbm.at[:, pl.ds(base, _SAMPLES_PER_W)],
                           pw_v, sem1)
    cp2 = pltpu.async_copy(frac_hbm.at[:, pl.ds(base, _SAMPLES_PER_W)],
                           frac_v, sem2)
    pltpu.sync_copy(table_hbm, table_v)

    iota = lax.iota(jnp.int32, _LANES)
    for k in range(256 // _LANES):
        i = k * _LANES + iota
        idx = jnp.minimum(i & 127, 119)
        mf = (i >> 7).astype(jnp.float32)
        t256_v[pl.ds(k * _LANES, _LANES)] = (
            plsc.load_gather(table_v, [idx]) * mf)

    cp1.wait()
    cp2.wait()

    def body(g, carry):
        loss_acc, abs_acc = carry
        accs = []
        for half in range(2):
            c0 = (g * 2 + half) * _LANES
            tc0 = jnp.zeros((_LANES,), jnp.float32)
            tc1 = jnp.zeros((_LANES,), jnp.float32)
            for l in range(0, _L, 2):
                w0 = pw_v[l, pl.ds(c0, _LANES)]
                f0 = frac_v[l, pl.ds(c0, _LANES)]
                w1 = pw_v[l + 1, pl.ds(c0, _LANES)]
                f1 = frac_v[l + 1, pl.ds(c0, _LANES)]
                tc0 = tc0 + f0 * plsc.load_gather(t256_v, [w0])
                tc1 = tc1 + f1 * plsc.load_gather(t256_v, [w1])
            tc = tc0 + tc1
            a = jnp.abs(tc)
            ex = jnp.maximum(a - _TOL, 0.0)
            e2 = jnp.exp(2.0 * ex)
            t = 1.0 - 2.0 / (e2 + 1.0)
            accs.append((t, a))
        loss_acc = loss_acc + accs[0][0] + accs[1][0]
        abs_acc = abs_acc + accs[0][1] + accs[1][1]
        return loss_acc, abs_acc

    zero = jnp.zeros((_LANES,), jnp.float32)
    loss_acc, abs_acc = lax.fori_loop(
        0, _SAMPLES_PER_W // (2 * _LANES), body, (zero, zero))

    out_v[0, :] = loss_acc * (1.0 / _B)
    out_v[1, :] = abs_acc * (1.0 / _B)
    pltpu.sync_copy(out_v, out_hbm.at[wid])


def kernel(element_indices, element_fractions, element_mask, oxidation_states):
    pw = (element_indices.astype(jnp.int32)
          | (element_mask.astype(jnp.int32) << 7)).T
    ef = element_fractions.T
    partials = _sc_charge_loss(pw, ef, oxidation_states)
    charge_balance_loss = jnp.sum(partials[:, 0, :])
    mean_charge_imbalance = jnp.sum(partials[:, 1, :])
    return (charge_balance_loss, mean_charge_imbalance)
